# baseline (device time: 79920 ns/iter reference)
import jax
import jax.numpy as jnp
from jax import lax
from jax.experimental import pallas as pl
from jax.experimental.pallas import tpu as pltpu

N_DEV = 16
BLK = 32


def kernel(x, Wq, Wo, K_ext, V_ext):
    B, Sq, D = x.shape
    _, Skv, Hkv, Dh = K_ext.shape
    Dq = Wq.shape[1]
    Hq = Dq // Dh
    G = Hq // Hkv
    Do = Wo.shape[1]
    R = B * Sq

    def body(x_ref, wq_ref, wo_ref, k_ref, v_ref, out_ref,
             o_acc, o_tx, o_rx, o_bf, l_acc, l_rx,
             ssem1, rsem1, ssem2, rsem2, lssem, lrsem):
        me = lax.axis_index("i")
        pending = []

        bar = pltpu.get_barrier_semaphore()
        for d in range(1, N_DEV):
            peer = lax.rem(me + d, N_DEV)
            pl.semaphore_signal(bar, inc=1, device_id=(peer,),
                                device_id_type=pl.DeviceIdType.MESH)
        pl.semaphore_wait(bar, N_DEV - 1)

        for b in range(B):
            xb = x_ref[b].astype(jnp.bfloat16)
            q = jnp.dot(xb, wq_ref[...].astype(jnp.bfloat16),
                        preferred_element_type=jnp.float32)
            kb = k_ref[b].reshape(Skv, Hkv * Dh).astype(jnp.bfloat16)
            vb = v_ref[b].reshape(Skv, Hkv * Dh).astype(jnp.bfloat16)
            for h in range(Hq):
                g = h // G
                qh = q[:, h * Dh:(h + 1) * Dh].astype(jnp.bfloat16)
                kh = kb[:, g * Dh:(g + 1) * Dh]
                vh = vb[:, g * Dh:(g + 1) * Dh]
                s = lax.dot_general(qh, kh, (((1,), (1,)), ((), ())),
                                    preferred_element_type=jnp.float32)
                p_ = jnp.exp(s * 0.125)
                l_acc[b * Sq:(b + 1) * Sq, h:h + 1] = jnp.sum(
                    p_, axis=1, keepdims=True)
                o_acc[b * Sq:(b + 1) * Sq, h * Dh:(h + 1) * Dh] = jnp.dot(
                    p_.astype(jnp.bfloat16), vh,
                    preferred_element_type=jnp.float32)

        o_tx[...] = o_acc[...].astype(jnp.bfloat16)

        p1_waits = []
        for d in range(1, N_DEV):
            peer = lax.rem(me + d, N_DEV)
            o_rdma = pltpu.make_async_remote_copy(
                src_ref=o_tx.at[pl.ds(peer * BLK, BLK)],
                dst_ref=o_rx.at[pl.ds(me * BLK, BLK)],
                send_sem=ssem1.at[d], recv_sem=rsem1.at[d],
                device_id=(peer,), device_id_type=pl.DeviceIdType.MESH)
            l_rdma = pltpu.make_async_remote_copy(
                src_ref=l_acc,
                dst_ref=l_rx.at[pl.ds(me * Sq * B, R)],
                send_sem=lssem.at[d], recv_sem=lrsem.at[d],
                device_id=(peer,), device_id_type=pl.DeviceIdType.MESH)
            o_rdma.start()
            l_rdma.start()
            pending.extend([o_rdma, l_rdma])
            src = lax.rem(me + N_DEV - d, N_DEV)
            o_wait = pltpu.make_async_remote_copy(
                src_ref=o_tx.at[pl.ds(peer * BLK, BLK)],
                dst_ref=o_rx.at[pl.ds(src * BLK, BLK)],
                send_sem=ssem1.at[d], recv_sem=rsem1.at[d],
                device_id=(peer,), device_id_type=pl.DeviceIdType.MESH)
            p1_waits.append(o_wait)

        o_rx[pl.ds(me * BLK, BLK), :] = o_tx[pl.ds(me * BLK, BLK), :]
        l_rx[pl.ds(me * R, R), :] = l_acc[...]
        for w in p1_waits:
            w.wait_recv()

        red = o_rx[0:BLK, :].astype(jnp.float32)
        for j in range(1, N_DEV):
            red = red + o_rx[j * BLK:(j + 1) * BLK, :].astype(jnp.float32)
        o_bf[pl.ds(me * BLK, BLK), :] = red.astype(jnp.bfloat16)

        p2_waits = []
        for d in range(1, N_DEV):
            peer = lax.rem(me + d, N_DEV)
            o_rdma = pltpu.make_async_remote_copy(
                src_ref=o_bf.at[pl.ds(me * BLK, BLK)],
                dst_ref=o_bf.at[pl.ds(me * BLK, BLK)],
                send_sem=ssem2.at[d], recv_sem=rsem2.at[d],
                device_id=(peer,), device_id_type=pl.DeviceIdType.MESH)
            o_rdma.start()
            pending.append(o_rdma)
            src = lax.rem(me + N_DEV - d, N_DEV)
            o_wait = pltpu.make_async_remote_copy(
                src_ref=o_bf.at[pl.ds(me * BLK, BLK)],
                dst_ref=o_bf.at[pl.ds(src * BLK, BLK)],
                send_sem=ssem2.at[d], recv_sem=rsem2.at[d],
                device_id=(peer,), device_id_type=pl.DeviceIdType.MESH)
            p2_waits.append(o_wait)

        for d in range(1, N_DEV):
            peer = lax.rem(me + d, N_DEV)
            src = lax.rem(me + N_DEV - d, N_DEV)
            l_wait = pltpu.make_async_remote_copy(
                src_ref=l_acc,
                dst_ref=l_rx.at[pl.ds(src * R, R)],
                send_sem=lssem.at[d], recv_sem=lrsem.at[d],
                device_id=(peer,), device_id_type=pl.DeviceIdType.MESH)
            l_wait.wait_recv()
        l_tot = l_rx[0:R, :]
        for j in range(1, N_DEV):
            l_tot = l_tot + l_rx[j * R:(j + 1) * R, :]

        for w in p2_waits:
            w.wait_recv()

        wo = wo_ref[...].astype(jnp.bfloat16)
        for b in range(B):
            o = o_bf[b * Sq:(b + 1) * Sq, :].astype(jnp.float32)
            blocks = []
            for h in range(Hq):
                lcol = l_tot[b * Sq:(b + 1) * Sq, h:h + 1]
                blocks.append(o[:, h * Dh:(h + 1) * Dh] / lcol)
            onorm = jnp.concatenate(blocks, axis=1).astype(jnp.bfloat16)
            out_ref[b] = jnp.dot(onorm, wo,
                                 preferred_element_type=jnp.float32)

        for dsc in pending:
            dsc.wait_send()

    return pl.pallas_call(
        body,
        out_shape=jax.ShapeDtypeStruct((B, Sq, Do), jnp.float32),
        in_specs=[pl.BlockSpec(memory_space=pltpu.VMEM)] * 5,
        out_specs=pl.BlockSpec(memory_space=pltpu.VMEM),
        scratch_shapes=[
            pltpu.VMEM((R, Dq), jnp.float32),
            pltpu.VMEM((R, Dq), jnp.bfloat16),
            pltpu.VMEM((N_DEV * BLK, Dq), jnp.bfloat16),
            pltpu.VMEM((R, Dq), jnp.bfloat16),
            pltpu.VMEM((R, Hq), jnp.float32),
            pltpu.VMEM((N_DEV * R, Hq), jnp.float32),
            pltpu.SemaphoreType.DMA((N_DEV,)),
            pltpu.SemaphoreType.DMA((N_DEV,)),
            pltpu.SemaphoreType.DMA((N_DEV,)),
            pltpu.SemaphoreType.DMA((N_DEV,)),
            pltpu.SemaphoreType.DMA((N_DEV,)),
            pltpu.SemaphoreType.DMA((N_DEV,)),
        ],
        compiler_params=pltpu.CompilerParams(collective_id=0),
    )(x, Wq, Wo, K_ext, V_ext)


# device time: 42407 ns/iter; 1.8846x vs baseline; 1.8846x over previous
import jax
import jax.numpy as jnp
from jax import lax
from jax.experimental import pallas as pl
from jax.experimental.pallas import tpu as pltpu

N_DEV = 16
LOG2_N = 4
BLK = 32


def kernel(x, Wq, Wo, K_ext, V_ext):
    B, Sq, D = x.shape
    _, Skv, Hkv, Dh = K_ext.shape
    Dq = Wq.shape[1]
    Hq = Dq // Dh
    G = Hq // Hkv
    Do = Wo.shape[1]
    R = B * Sq
    W = Dq + Hq
    OFF = [0, 256, 384, 448]

    def body(x_ref, wq_ref, wo_ref, k_ref, v_ref, out_ref,
             o_acc, o_tx, o_rx, out_bf, o_ssem, o_rsem):
        me = lax.axis_index("i")
        pos = (((me & 1) << 3) | ((me & 2) << 1)
               | ((me & 4) >> 1) | ((me & 8) >> 3))
        even = (me & 1) == 0
        pending = []

        bar = pltpu.get_barrier_semaphore()
        for r in range(LOG2_N):
            p = jnp.bitwise_xor(me, 1 << r)
            pl.semaphore_signal(bar, inc=1, device_id=(p,),
                                device_id_type=pl.DeviceIdType.MESH)
        pl.semaphore_wait(bar, LOG2_N)

        def compute_partial(b):
            xb = x_ref[b].astype(jnp.bfloat16)
            q = jnp.dot(xb, wq_ref[...].astype(jnp.bfloat16),
                        preferred_element_type=jnp.float32)
            kb = k_ref[b].reshape(Skv, Hkv * Dh).astype(jnp.bfloat16)
            vb = v_ref[b].reshape(Skv, Hkv * Dh).astype(jnp.bfloat16)
            for h in range(Hq):
                g = h // G
                qh = q[:, h * Dh:(h + 1) * Dh].astype(jnp.bfloat16)
                kh = kb[:, g * Dh:(g + 1) * Dh]
                vh = vb[:, g * Dh:(g + 1) * Dh]
                s = lax.dot_general(qh, kh, (((1,), (1,)), ((), ())),
                                    preferred_element_type=jnp.float32)
                p_ = jnp.exp(s * 0.125)
                o_acc[b * Sq:(b + 1) * Sq, Dq + h:Dq + h + 1] = jnp.sum(
                    p_, axis=1, keepdims=True)
                o_acc[b * Sq:(b + 1) * Sq, h * Dh:(h + 1) * Dh] = jnp.dot(
                    p_.astype(jnp.bfloat16), vh,
                    preferred_element_type=jnp.float32)

        def rs_round(k):
            p = jnp.bitwise_xor(me, 1 << k)
            nblk = 8 >> k
            rows = nblk * BLK
            s_keep = (pos >> (3 - k)) << (3 - k)
            s_send = jnp.bitwise_xor(s_keep, nblk)
            o_tx[OFF[k]:OFF[k] + rows, :] = o_acc[
                pl.ds(s_send * BLK, rows), :].astype(jnp.bfloat16)
            o_rdma = pltpu.make_async_remote_copy(
                src_ref=o_tx.at[pl.ds(OFF[k], rows)],
                dst_ref=o_rx.at[pl.ds(OFF[k], rows)],
                send_sem=o_ssem.at[k], recv_sem=o_rsem.at[k],
                device_id=(p,), device_id_type=pl.DeviceIdType.MESH)
            o_rdma.start()
            pending.append(o_rdma)
            return o_rdma, rows, s_keep, OFF[k]

        def rs_finish(o_rdma, rows, s_keep, off):
            o_rdma.wait_recv()
            o_acc[pl.ds(s_keep * BLK, rows), :] = (
                o_acc[pl.ds(s_keep * BLK, rows), :]
                + o_rx[pl.ds(off, rows), :].astype(jnp.float32))

        @pl.when(even)
        def _():
            compute_partial(1)

        @pl.when(jnp.logical_not(even))
        def _():
            compute_partial(0)

        r0 = rs_round(0)

        @pl.when(even)
        def _():
            compute_partial(0)

        @pl.when(jnp.logical_not(even))
        def _():
            compute_partial(1)

        rs_finish(*r0)
        for k in range(1, LOG2_N):
            rs_finish(*rs_round(k))

        o_red = o_acc[pl.ds(pos * BLK, BLK), :]
        blocks = []
        for h in range(Hq):
            lcol = o_red[:, Dq + h:Dq + h + 1]
            blocks.append(o_red[:, h * Dh:(h + 1) * Dh] / lcol)
        onorm = jnp.concatenate(blocks, axis=1).astype(jnp.bfloat16)
        myout = jnp.dot(onorm, wo_ref[...].astype(jnp.bfloat16),
                        preferred_element_type=jnp.float32)
        out_bf[pl.ds(pos * BLK, BLK), :] = myout.astype(jnp.bfloat16)

        for k in range(LOG2_N):
            p = jnp.bitwise_xor(me, 1 << (3 - k))
            nblk = 1 << k
            rows = nblk * BLK
            s_mine = (pos >> k) << k
            o_rdma = pltpu.make_async_remote_copy(
                src_ref=out_bf.at[pl.ds(s_mine * BLK, rows)],
                dst_ref=out_bf.at[pl.ds(s_mine * BLK, rows)],
                send_sem=o_ssem.at[LOG2_N + k], recv_sem=o_rsem.at[LOG2_N + k],
                device_id=(p,), device_id_type=pl.DeviceIdType.MESH)
            o_rdma.start()
            pending.append(o_rdma)
            o_rdma.wait_recv()

        out_ref[...] = out_bf[...].astype(jnp.float32)

        for dsc in pending:
            dsc.wait_send()

    out_flat = pl.pallas_call(
        body,
        out_shape=jax.ShapeDtypeStruct((R, Do), jnp.float32),
        in_specs=[pl.BlockSpec(memory_space=pltpu.VMEM)] * 5,
        out_specs=pl.BlockSpec(memory_space=pltpu.VMEM),
        scratch_shapes=[
            pltpu.VMEM((R, W), jnp.float32),
            pltpu.VMEM((480, W), jnp.bfloat16),
            pltpu.VMEM((480, W), jnp.bfloat16),
            pltpu.VMEM((R, Do), jnp.bfloat16),
            pltpu.SemaphoreType.DMA((2 * LOG2_N,)),
            pltpu.SemaphoreType.DMA((2 * LOG2_N,)),
        ],
        compiler_params=pltpu.CompilerParams(collective_id=0),
    )(x, Wq, Wo, K_ext, V_ext)
    return out_flat.reshape(B, Sq, Do)


# device time: 42171 ns/iter; 1.8951x vs baseline; 1.0056x over previous
import jax
import jax.numpy as jnp
from jax import lax
from jax.experimental import pallas as pl
from jax.experimental.pallas import tpu as pltpu

N_DEV = 16
LOG2_N = 4
BLK = 32


def kernel(x, Wq, Wo, K_ext, V_ext):
    B, Sq, D = x.shape
    _, Skv, Hkv, Dh = K_ext.shape
    Dq = Wq.shape[1]
    Hq = Dq // Dh
    G = Hq // Hkv
    Do = Wo.shape[1]
    R = B * Sq
    W = Dq + Hq
    OFF = [0, 256, 384, 448]

    def body(x_ref, wq_ref, wo_ref, k_ref, v_ref, out_ref,
             o_acc, o_tx, o_rx, out_bf, o_ssem, o_rsem):
        me = lax.axis_index("i")
        pos = (((me & 1) << 3) | ((me & 2) << 1)
               | ((me & 4) >> 1) | ((me & 8) >> 3))
        even = (me & 1) == 0
        pending = []

        bar = pltpu.get_barrier_semaphore()
        for r in range(LOG2_N):
            p = jnp.bitwise_xor(me, 1 << r)
            pl.semaphore_signal(bar, inc=1, device_id=(p,),
                                device_id_type=pl.DeviceIdType.MESH)
        pl.semaphore_wait(bar, LOG2_N)

        def compute_partial(b):
            xb = x_ref[b].astype(jnp.bfloat16)
            q = jnp.dot(xb, wq_ref[...].astype(jnp.bfloat16),
                        preferred_element_type=jnp.float32)
            kb = k_ref[b].reshape(Skv, Hkv * Dh).astype(jnp.bfloat16)
            vb = v_ref[b].reshape(Skv, Hkv * Dh).astype(jnp.bfloat16)
            for g in range(Hkv):
                qg = jnp.concatenate(
                    [q[:, (g * G + i) * Dh:(g * G + i + 1) * Dh]
                     for i in range(G)], axis=0).astype(jnp.bfloat16)
                kh = kb[:, g * Dh:(g + 1) * Dh]
                vh = vb[:, g * Dh:(g + 1) * Dh]
                s = lax.dot_general(qg, kh, (((1,), (1,)), ((), ())),
                                    preferred_element_type=jnp.float32)
                p_ = jnp.exp(s * 0.125)
                lsum = jnp.sum(p_, axis=1, keepdims=True)
                og = jnp.dot(p_.astype(jnp.bfloat16), vh,
                             preferred_element_type=jnp.float32)
                for i in range(G):
                    h = g * G + i
                    o_acc[b * Sq:(b + 1) * Sq, Dq + h:Dq + h + 1] = lsum[
                        i * Sq:(i + 1) * Sq, :]
                    o_acc[b * Sq:(b + 1) * Sq, h * Dh:(h + 1) * Dh] = og[
                        i * Sq:(i + 1) * Sq, :]

        def rs_round(k):
            p = jnp.bitwise_xor(me, 1 << k)
            nblk = 8 >> k
            rows = nblk * BLK
            s_keep = (pos >> (3 - k)) << (3 - k)
            s_send = jnp.bitwise_xor(s_keep, nblk)
            o_tx[OFF[k]:OFF[k] + rows, :] = o_acc[
                pl.ds(s_send * BLK, rows), :].astype(jnp.bfloat16)
            o_rdma = pltpu.make_async_remote_copy(
                src_ref=o_tx.at[pl.ds(OFF[k], rows)],
                dst_ref=o_rx.at[pl.ds(OFF[k], rows)],
                send_sem=o_ssem.at[k], recv_sem=o_rsem.at[k],
                device_id=(p,), device_id_type=pl.DeviceIdType.MESH)
            o_rdma.start()
            pending.append(o_rdma)
            return o_rdma, rows, s_keep, OFF[k]

        def rs_finish(o_rdma, rows, s_keep, off):
            o_rdma.wait_recv()
            o_acc[pl.ds(s_keep * BLK, rows), :] = (
                o_acc[pl.ds(s_keep * BLK, rows), :]
                + o_rx[pl.ds(off, rows), :].astype(jnp.float32))

        @pl.when(even)
        def _():
            compute_partial(1)

        @pl.when(jnp.logical_not(even))
        def _():
            compute_partial(0)

        r0 = rs_round(0)

        @pl.when(even)
        def _():
            compute_partial(0)

        @pl.when(jnp.logical_not(even))
        def _():
            compute_partial(1)

        rs_finish(*r0)
        for k in range(1, LOG2_N):
            rs_finish(*rs_round(k))

        o_red = o_acc[pl.ds(pos * BLK, BLK), :]
        blocks = []
        for h in range(Hq):
            lcol = o_red[:, Dq + h:Dq + h + 1]
            blocks.append(o_red[:, h * Dh:(h + 1) * Dh] / lcol)
        onorm = jnp.concatenate(blocks, axis=1).astype(jnp.bfloat16)
        myout = jnp.dot(onorm, wo_ref[...].astype(jnp.bfloat16),
                        preferred_element_type=jnp.float32)
        out_bf[pl.ds(pos * BLK, BLK), :] = myout.astype(jnp.bfloat16)

        for k in range(LOG2_N):
            p = jnp.bitwise_xor(me, 1 << (3 - k))
            nblk = 1 << k
            rows = nblk * BLK
            s_mine = (pos >> k) << k
            o_rdma = pltpu.make_async_remote_copy(
                src_ref=out_bf.at[pl.ds(s_mine * BLK, rows)],
                dst_ref=out_bf.at[pl.ds(s_mine * BLK, rows)],
                send_sem=o_ssem.at[LOG2_N + k], recv_sem=o_rsem.at[LOG2_N + k],
                device_id=(p,), device_id_type=pl.DeviceIdType.MESH)
            o_rdma.start()
            pending.append(o_rdma)
            o_rdma.wait_recv()

        out_ref[...] = out_bf[...].astype(jnp.float32)

        for dsc in pending:
            dsc.wait_send()

    out_flat = pl.pallas_call(
        body,
        out_shape=jax.ShapeDtypeStruct((R, Do), jnp.float32),
        in_specs=[pl.BlockSpec(memory_space=pltpu.VMEM)] * 5,
        out_specs=pl.BlockSpec(memory_space=pltpu.VMEM),
        scratch_shapes=[
            pltpu.VMEM((R, W), jnp.float32),
            pltpu.VMEM((480, W), jnp.bfloat16),
            pltpu.VMEM((480, W), jnp.bfloat16),
            pltpu.VMEM((R, Do), jnp.bfloat16),
            pltpu.SemaphoreType.DMA((2 * LOG2_N,)),
            pltpu.SemaphoreType.DMA((2 * LOG2_N,)),
        ],
        compiler_params=pltpu.CompilerParams(collective_id=0),
    )(x, Wq, Wo, K_ext, V_ext)
    return out_flat.reshape(B, Sq, Do)


# device time: 39497 ns/iter; 2.0234x vs baseline; 1.0677x over previous
import jax
import jax.numpy as jnp
from jax import lax
from jax.experimental import pallas as pl
from jax.experimental.pallas import tpu as pltpu

N_DEV = 16
LOG2_N = 4
BLK = 32


def kernel(x, Wq, Wo, K_ext, V_ext):
    B, Sq, D = x.shape
    _, Skv, Hkv, Dh = K_ext.shape
    Dq = Wq.shape[1]
    Hq = Dq // Dh
    G = Hq // Hkv
    Do = Wo.shape[1]
    R = B * Sq
    W = Dq + Hq
    OFF = [0, 256, 384, 448]

    def body(x_ref, wq_ref, wo_ref, k_ref, v_ref, out_ref,
             o_acc, o_tx, o_rx, out_bf, o_ssem, o_rsem):
        me = lax.axis_index("i")
        pos = (((me & 1) << 3) | ((me & 2) << 1)
               | ((me & 4) >> 1) | ((me & 8) >> 3))
        even = (me & 1) == 0
        pending = []

        bar = pltpu.get_barrier_semaphore()
        for r in range(LOG2_N):
            p = jnp.bitwise_xor(me, 1 << r)
            pl.semaphore_signal(bar, inc=1, device_id=(p,),
                                device_id_type=pl.DeviceIdType.MESH)
        pl.semaphore_wait(bar, LOG2_N)

        def compute_partial(b):
            xb = x_ref[b].astype(jnp.bfloat16)
            q = jnp.dot(xb, wq_ref[...].astype(jnp.bfloat16),
                        preferred_element_type=jnp.float32)
            kb = k_ref[b].reshape(Skv, Hkv * Dh).astype(jnp.bfloat16)
            vb = v_ref[b].reshape(Skv, Hkv * Dh).astype(jnp.bfloat16)
            for g in range(Hkv):
                qg = jnp.concatenate(
                    [q[:, (g * G + i) * Dh:(g * G + i + 1) * Dh]
                     for i in range(G)], axis=0).astype(jnp.bfloat16)
                kh = kb[:, g * Dh:(g + 1) * Dh]
                vh = vb[:, g * Dh:(g + 1) * Dh]
                s = lax.dot_general(qg, kh, (((1,), (1,)), ((), ())),
                                    preferred_element_type=jnp.float32)
                p_ = jnp.exp(s * 0.125)
                lsum = jnp.sum(p_, axis=1, keepdims=True)
                og = jnp.dot(p_.astype(jnp.bfloat16), vh,
                             preferred_element_type=jnp.float32)
                for i in range(G):
                    h = g * G + i
                    o_acc[b * Sq:(b + 1) * Sq, Dq + h:Dq + h + 1] = lsum[
                        i * Sq:(i + 1) * Sq, :]
                    o_acc[b * Sq:(b + 1) * Sq, h * Dh:(h + 1) * Dh] = og[
                        i * Sq:(i + 1) * Sq, :]

        def rs_round(k):
            p = jnp.bitwise_xor(me, 1 << k)
            nblk = 8 >> k
            rows = nblk * BLK
            s_keep = (pos >> (3 - k)) << (3 - k)
            s_send = jnp.bitwise_xor(s_keep, nblk)
            o_tx[OFF[k]:OFF[k] + rows, :] = o_acc[
                pl.ds(s_send * BLK, rows), :].astype(jnp.bfloat16)
            o_rdma = pltpu.make_async_remote_copy(
                src_ref=o_tx.at[pl.ds(OFF[k], rows)],
                dst_ref=o_rx.at[pl.ds(OFF[k], rows)],
                send_sem=o_ssem.at[k], recv_sem=o_rsem.at[k],
                device_id=(p,), device_id_type=pl.DeviceIdType.MESH)
            o_rdma.start()
            pending.append(o_rdma)
            return o_rdma, rows, s_keep, OFF[k]

        def rs_finish(o_rdma, rows, s_keep, off):
            o_rdma.wait_recv()
            o_acc[pl.ds(s_keep * BLK, rows), :] = (
                o_acc[pl.ds(s_keep * BLK, rows), :]
                + o_rx[pl.ds(off, rows), :].astype(jnp.float32))

        @pl.when(even)
        def _():
            compute_partial(1)

        @pl.when(jnp.logical_not(even))
        def _():
            compute_partial(0)

        r0 = rs_round(0)

        @pl.when(even)
        def _():
            compute_partial(0)

        @pl.when(jnp.logical_not(even))
        def _():
            compute_partial(1)

        rs_finish(*r0)
        for k in range(1, LOG2_N):
            rs_finish(*rs_round(k))

        o_red = o_acc[pl.ds(pos * BLK, BLK), :]
        blocks = []
        for h in range(Hq):
            lcol = o_red[:, Dq + h:Dq + h + 1]
            blocks.append(o_red[:, h * Dh:(h + 1) * Dh] / lcol)
        onorm = jnp.concatenate(blocks, axis=1).astype(jnp.bfloat16)
        myout = jnp.dot(onorm, wo_ref[...].astype(jnp.bfloat16),
                        preferred_element_type=jnp.float32)
        out_bf[pl.ds(pos * BLK, BLK), :] = myout.astype(jnp.bfloat16)

        def ag_send(region_start, nblk, partner, sem_idx):
            rdma = pltpu.make_async_remote_copy(
                src_ref=out_bf.at[pl.ds(region_start * BLK, nblk * BLK)],
                dst_ref=out_bf.at[pl.ds(region_start * BLK, nblk * BLK)],
                send_sem=o_ssem.at[sem_idx], recv_sem=o_rsem.at[sem_idx],
                device_id=(partner,), device_id_type=pl.DeviceIdType.MESH)
            rdma.start()
            pending.append(rdma)
            return rdma

        def win(k):
            return (pos >> k) << k

        partners = [jnp.bitwise_xor(me, 1 << (3 - k)) for k in range(LOG2_N)]
        r0 = ag_send(win(0), 1, partners[0], LOG2_N)
        p1 = {1: ag_send(win(0), 1, partners[1], LOG2_N + 1)}
        p2 = {}
        r0.wait_recv()
        p2[1] = ag_send(jnp.bitwise_xor(win(0), 1), 1, partners[1],
                        2 * LOG2_N + 1)
        p1[2] = ag_send(win(1), 2, partners[2], LOG2_N + 2)
        p1[1].wait_recv()
        p2[1].wait_recv()
        p2[2] = ag_send(jnp.bitwise_xor(win(1), 2), 2, partners[2],
                        2 * LOG2_N + 2)
        p1[3] = ag_send(win(2), 4, partners[3], LOG2_N + 3)
        p1[2].wait_recv()
        p2[2].wait_recv()
        p2[3] = ag_send(jnp.bitwise_xor(win(2), 4), 4, partners[3],
                        2 * LOG2_N + 3)
        p1[3].wait_recv()
        p2[3].wait_recv()

        out_ref[...] = out_bf[...].astype(jnp.float32)

        for dsc in pending:
            dsc.wait_send()

    out_flat = pl.pallas_call(
        body,
        out_shape=jax.ShapeDtypeStruct((R, Do), jnp.float32),
        in_specs=[pl.BlockSpec(memory_space=pltpu.VMEM)] * 5,
        out_specs=pl.BlockSpec(memory_space=pltpu.VMEM),
        scratch_shapes=[
            pltpu.VMEM((R, W), jnp.float32),
            pltpu.VMEM((480, W), jnp.bfloat16),
            pltpu.VMEM((480, W), jnp.bfloat16),
            pltpu.VMEM((R, Do), jnp.bfloat16),
            pltpu.SemaphoreType.DMA((3 * LOG2_N,)),
            pltpu.SemaphoreType.DMA((3 * LOG2_N,)),
        ],
        compiler_params=pltpu.CompilerParams(collective_id=0),
    )(x, Wq, Wo, K_ext, V_ext)
    return out_flat.reshape(B, Sq, Do)


# device time: 37234 ns/iter; 2.1464x vs baseline; 1.0608x over previous
import jax
import jax.numpy as jnp
from jax import lax
from jax.experimental import pallas as pl
from jax.experimental.pallas import tpu as pltpu

N_DEV = 16
LOG2_N = 4
BLK = 32


def kernel(x, Wq, Wo, K_ext, V_ext):
    B, Sq, D = x.shape
    _, Skv, Hkv, Dh = K_ext.shape
    Dq = Wq.shape[1]
    Hq = Dq // Dh
    G = Hq // Hkv
    Do = Wo.shape[1]
    R = B * Sq
    W = Dq + Hq
    OFF = [0, 256, 384, 448]

    def body(x_ref, wq_ref, wo_ref, k_ref, v_ref, out_ref,
             o_acc, o_tx, o_rx, out_bf, o_ssem, o_rsem):
        me = lax.axis_index("i")
        pos = (((me & 1) << 3) | ((me & 2) << 1)
               | ((me & 4) >> 1) | ((me & 8) >> 3))
        even = (me & 1) == 0
        pending = []

        def compute_partial(b):
            xb = x_ref[b].astype(jnp.bfloat16)
            q = jnp.dot(xb, wq_ref[...].astype(jnp.bfloat16),
                        preferred_element_type=jnp.float32) * 0.125
            kb = k_ref[b].reshape(Skv, Hkv * Dh).astype(jnp.bfloat16)
            vb = v_ref[b].reshape(Skv, Hkv * Dh).astype(jnp.bfloat16)
            for g in range(Hkv):
                qg = jnp.concatenate(
                    [q[:, (g * G + i) * Dh:(g * G + i + 1) * Dh]
                     for i in range(G)], axis=0).astype(jnp.bfloat16)
                kh = kb[:, g * Dh:(g + 1) * Dh]
                vh = vb[:, g * Dh:(g + 1) * Dh]
                s = lax.dot_general(qg, kh, (((1,), (1,)), ((), ())),
                                    preferred_element_type=jnp.float32)
                p_ = jnp.exp(s)
                lsum = jnp.sum(p_, axis=1, keepdims=True)
                og = jnp.dot(p_.astype(jnp.bfloat16), vh,
                             preferred_element_type=jnp.float32)
                for i in range(G):
                    h = g * G + i
                    o_acc[b * Sq:(b + 1) * Sq, Dq + h:Dq + h + 1] = lsum[
                        i * Sq:(i + 1) * Sq, :]
                    o_acc[b * Sq:(b + 1) * Sq, h * Dh:(h + 1) * Dh] = og[
                        i * Sq:(i + 1) * Sq, :]

        def rs_round(k):
            p = jnp.bitwise_xor(me, 1 << k)
            nblk = 8 >> k
            rows = nblk * BLK
            s_keep = (pos >> (3 - k)) << (3 - k)
            s_send = jnp.bitwise_xor(s_keep, nblk)
            o_tx[OFF[k]:OFF[k] + rows, :] = o_acc[
                pl.ds(s_send * BLK, rows), :].astype(jnp.bfloat16)
            o_rdma = pltpu.make_async_remote_copy(
                src_ref=o_tx.at[pl.ds(OFF[k], rows)],
                dst_ref=o_rx.at[pl.ds(OFF[k], rows)],
                send_sem=o_ssem.at[k], recv_sem=o_rsem.at[k],
                device_id=(p,), device_id_type=pl.DeviceIdType.MESH)
            o_rdma.start()
            pending.append(o_rdma)
            return o_rdma, rows, s_keep, OFF[k]

        def rs_finish(o_rdma, rows, s_keep, off):
            o_rdma.wait_recv()
            o_acc[pl.ds(s_keep * BLK, rows), :] = (
                o_acc[pl.ds(s_keep * BLK, rows), :]
                + o_rx[pl.ds(off, rows), :].astype(jnp.float32))

        @pl.when(even)
        def _():
            compute_partial(1)

        @pl.when(jnp.logical_not(even))
        def _():
            compute_partial(0)

        bar = pltpu.get_barrier_semaphore()
        for r in range(LOG2_N):
            p = jnp.bitwise_xor(me, 1 << r)
            pl.semaphore_signal(bar, inc=1, device_id=(p,),
                                device_id_type=pl.DeviceIdType.MESH)
        pl.semaphore_wait(bar, LOG2_N)

        r0 = rs_round(0)

        @pl.when(even)
        def _():
            compute_partial(0)

        @pl.when(jnp.logical_not(even))
        def _():
            compute_partial(1)

        rs_finish(*r0)
        for k in range(1, LOG2_N):
            rs_finish(*rs_round(k))

        o_red = o_acc[pl.ds(pos * BLK, BLK), :]
        blocks = []
        for h in range(Hq):
            lcol = o_red[:, Dq + h:Dq + h + 1]
            blocks.append(o_red[:, h * Dh:(h + 1) * Dh] / lcol)
        onorm = jnp.concatenate(blocks, axis=1).astype(jnp.bfloat16)
        myout = jnp.dot(onorm, wo_ref[...].astype(jnp.bfloat16),
                        preferred_element_type=jnp.float32)
        out_bf[pl.ds(pos * BLK, BLK), :] = myout.astype(jnp.bfloat16)

        def ag_send(region_start, nblk, partner, sem_idx):
            rdma = pltpu.make_async_remote_copy(
                src_ref=out_bf.at[pl.ds(region_start * BLK, nblk * BLK)],
                dst_ref=out_bf.at[pl.ds(region_start * BLK, nblk * BLK)],
                send_sem=o_ssem.at[sem_idx], recv_sem=o_rsem.at[sem_idx],
                device_id=(partner,), device_id_type=pl.DeviceIdType.MESH)
            rdma.start()
            pending.append(rdma)
            return rdma

        def win(k):
            return (pos >> k) << k

        partners = [jnp.bitwise_xor(me, 1 << (3 - k)) for k in range(LOG2_N)]
        r0 = ag_send(win(0), 1, partners[0], LOG2_N)
        p1 = {1: ag_send(win(0), 1, partners[1], LOG2_N + 1)}
        p2 = {}
        r0.wait_recv()
        p2[1] = ag_send(jnp.bitwise_xor(win(0), 1), 1, partners[1],
                        2 * LOG2_N + 1)
        p1[2] = ag_send(win(1), 2, partners[2], LOG2_N + 2)
        p1[1].wait_recv()
        p2[1].wait_recv()
        p2[2] = ag_send(jnp.bitwise_xor(win(1), 2), 2, partners[2],
                        2 * LOG2_N + 2)
        p1[3] = ag_send(win(2), 4, partners[3], LOG2_N + 3)
        p1[2].wait_recv()
        p2[2].wait_recv()
        p2[3] = ag_send(jnp.bitwise_xor(win(2), 4), 4, partners[3],
                        2 * LOG2_N + 3)
        p1[3].wait_recv()
        p2[3].wait_recv()

        out_ref[...] = out_bf[...].astype(jnp.float32)

        for dsc in pending:
            dsc.wait_send()

    out_flat = pl.pallas_call(
        body,
        out_shape=jax.ShapeDtypeStruct((R, Do), jnp.float32),
        in_specs=[pl.BlockSpec(memory_space=pltpu.VMEM)] * 5,
        out_specs=pl.BlockSpec(memory_space=pltpu.VMEM),
        scratch_shapes=[
            pltpu.VMEM((R, W), jnp.float32),
            pltpu.VMEM((480, W), jnp.bfloat16),
            pltpu.VMEM((480, W), jnp.bfloat16),
            pltpu.VMEM((R, Do), jnp.bfloat16),
            pltpu.SemaphoreType.DMA((3 * LOG2_N,)),
            pltpu.SemaphoreType.DMA((3 * LOG2_N,)),
        ],
        compiler_params=pltpu.CompilerParams(collective_id=0),
    )(x, Wq, Wo, K_ext, V_ext)
    return out_flat.reshape(B, Sq, Do)
